# Initial kernel scaffold; baseline (speedup 1.0000x reference)
import jax
import jax.numpy as jnp
from jax.experimental import pallas as pl


def kernel(tokens, emb_table, W, b):
    raise NotImplementedError("write your pallas kernel here")


# speedup vs baseline: 1.5733x; 1.5733x over previous
import functools

import jax
import jax.numpy as jnp
from jax import lax
from jax.experimental import pallas as pl
from jax.experimental.pallas import tpu as pltpu
from jax.experimental.pallas import tpu_sc as plsc

VOCAB = 1000
EMBED = 128
BATCH = 16384
SEQ = 20

NC = 2
NS = 16
LANES = 16

FHALF = EMBED // NC
ROWS_W = BATCH // NS
GROUPS = ROWS_W // LANES
GBLK = 8
STAGE = GBLK * LANES * FHALF


def _fold_body(emb_ref, w_ref, b_ref, out_ref):
    tbl = emb_ref[...]
    rid = lax.broadcasted_iota(jnp.int32, tbl.shape, 0)
    tbl = jnp.where(rid == 0, 0.0, tbl)
    t2 = lax.dot_general(tbl, w_ref[...], (((1,), (1,)), ((), ())),
                         preferred_element_type=jnp.float32)
    out_ref[...] = (t2 + b_ref[...]) * (1.0 / SEQ)


_fold = pl.pallas_call(
    _fold_body,
    out_shape=jax.ShapeDtypeStruct((VOCAB, EMBED), jnp.float32),
)


def _gather_body(t2_hbm, tok_hbm, out_hbm, table_v, tok_v, stage_v):
    c = lax.axis_index("c")
    s = lax.axis_index("s")
    pltpu.sync_copy(t2_hbm.at[c], table_v)
    pltpu.sync_copy(tok_hbm.at[s], tok_v)

    sidx0 = lax.iota(jnp.int32, LANES) * FHALF

    def t_body(t, carry):
        for gg in range(GBLK):
            g = t * GBLK + gg
            tb = [tok_v[l, pl.ds(g * LANES, LANES)] * FHALF
                  for l in range(SEQ)]

            def f_body(f, carry2, tb=tb, gg=gg):
                vals = [plsc.load_gather(table_v, [tb[l] + f])
                        for l in range(SEQ)]
                while len(vals) > 1:
                    nxt = [vals[i] + vals[i + 1]
                           for i in range(0, len(vals) - 1, 2)]
                    if len(vals) % 2:
                        nxt.append(vals[-1])
                    vals = nxt
                plsc.store_scatter(stage_v,
                                   [sidx0 + (gg * LANES * FHALF + f)],
                                   vals[0])
                return carry2

            lax.fori_loop(0, FHALF, f_body, 0)
        off = c * (BATCH * FHALF) + s * (ROWS_W * FHALF) + t * STAGE
        pltpu.sync_copy(stage_v, out_hbm.at[pl.ds(off, STAGE)])
        return carry

    lax.fori_loop(0, GROUPS // GBLK, t_body, 0)


_gather = functools.partial(
    pl.kernel,
    out_type=jax.ShapeDtypeStruct((NC * BATCH * FHALF,), jnp.float32),
    mesh=plsc.VectorSubcoreMesh(core_axis_name="c", subcore_axis_name="s",
                                num_cores=NC, num_subcores=NS),
    scratch_types=[
        pltpu.VMEM((VOCAB * FHALF,), jnp.float32),
        pltpu.VMEM((SEQ, ROWS_W), jnp.int32),
        pltpu.VMEM((STAGE,), jnp.float32),
    ],
    compiler_params=pltpu.CompilerParams(needs_layout_passes=False),
)(_gather_body)


def kernel(tokens, emb_table, W, b):
    t2b = _fold(emb_table, W, b.reshape(1, EMBED))
    t2_parts = t2b.reshape(VOCAB, NC, FHALF).transpose(1, 0, 2)
    t2_parts = t2_parts.reshape(NC, VOCAB * FHALF)
    tok = tokens.astype(jnp.int32).T.reshape(SEQ, NS, ROWS_W)
    tok = tok.transpose(1, 0, 2)
    out = _gather(t2_parts, tok)
    out = out.reshape(NC, BATCH, FHALF).transpose(1, 0, 2)
    return out.reshape(BATCH, EMBED)


# speedup vs baseline: 7.1169x; 4.5235x over previous
import functools

import jax
import jax.numpy as jnp
from jax import lax
from jax.experimental import pallas as pl
from jax.experimental.pallas import tpu as pltpu
from jax.experimental.pallas import tpu_sc as plsc

VOCAB = 1000
EMBED = 128
BATCH = 16384
SEQ = 20

NC = 2
NS = 16
LANES = 16

FHALF = EMBED // NC
FPAD = FHALF + 1
ROWS_W = BATCH // NS
GROUPS = ROWS_W // LANES
GBLK = 8
STAGE = GBLK * LANES * FHALF


def _fold_body(emb_ref, w_ref, b_ref, out_ref):
    tbl = emb_ref[...]
    rid = lax.broadcasted_iota(jnp.int32, tbl.shape, 0)
    tbl = jnp.where(rid == 0, 0.0, tbl)
    t2 = lax.dot_general(tbl, w_ref[...], (((1,), (1,)), ((), ())),
                         preferred_element_type=jnp.float32)
    out_ref[...] = (t2 + b_ref[...]) * (1.0 / SEQ)


_fold = pl.pallas_call(
    _fold_body,
    out_shape=jax.ShapeDtypeStruct((VOCAB, EMBED), jnp.float32),
)


def _gather_body(t2_hbm, tok_hbm, out_hbm, table_v, tok_v, stage_v):
    c = lax.axis_index("c")
    s = lax.axis_index("s")
    pltpu.sync_copy(t2_hbm.at[c], table_v)
    pltpu.sync_copy(tok_hbm.at[s], tok_v)

    sidx0 = lax.iota(jnp.int32, LANES) * FHALF

    def t_body(t, carry):
        for gg in range(GBLK):
            g = t * GBLK + gg
            tb = [tok_v[l, pl.ds(g * LANES, LANES)] * FPAD
                  for l in range(SEQ)]

            def f_body(f, carry2, tb=tb, gg=gg):
                vals = [plsc.load_gather(table_v, [tb[l] + f])
                        for l in range(SEQ)]
                while len(vals) > 1:
                    nxt = [vals[i] + vals[i + 1]
                           for i in range(0, len(vals) - 1, 2)]
                    if len(vals) % 2:
                        nxt.append(vals[-1])
                    vals = nxt
                plsc.store_scatter(stage_v,
                                   [sidx0 + (gg * LANES * FHALF + f)],
                                   vals[0])
                return carry2

            lax.fori_loop(0, FHALF, f_body, 0)
        off = c * (BATCH * FHALF) + s * (ROWS_W * FHALF) + t * STAGE
        pltpu.sync_copy(stage_v, out_hbm.at[pl.ds(off, STAGE)])
        return carry

    lax.fori_loop(0, GROUPS // GBLK, t_body, 0)


_gather = functools.partial(
    pl.kernel,
    out_type=jax.ShapeDtypeStruct((NC * BATCH * FHALF,), jnp.float32),
    mesh=plsc.VectorSubcoreMesh(core_axis_name="c", subcore_axis_name="s",
                                num_cores=NC, num_subcores=NS),
    scratch_types=[
        pltpu.VMEM((VOCAB * FPAD,), jnp.float32),
        pltpu.VMEM((SEQ, ROWS_W), jnp.int32),
        pltpu.VMEM((STAGE,), jnp.float32),
    ],
    compiler_params=pltpu.CompilerParams(needs_layout_passes=False),
)(_gather_body)


def kernel(tokens, emb_table, W, b):
    t2b = _fold(emb_table, W, b.reshape(1, EMBED))
    t2_parts = t2b.reshape(VOCAB, NC, FHALF).transpose(1, 0, 2)
    t2_parts = jnp.pad(t2_parts, ((0, 0), (0, 0), (0, FPAD - FHALF)))
    t2_parts = t2_parts.reshape(NC, VOCAB * FPAD)
    tok = tokens.astype(jnp.int32).T.reshape(SEQ, NS, ROWS_W)
    tok = tok.transpose(1, 0, 2)
    out = _gather(t2_parts, tok)
    out = out.reshape(NC, BATCH, FHALF).transpose(1, 0, 2)
    return out.reshape(BATCH, EMBED)


# speedup vs baseline: 10.2627x; 1.4420x over previous
import functools

import jax
import jax.numpy as jnp
from jax import lax
from jax.experimental import pallas as pl
from jax.experimental.pallas import tpu as pltpu
from jax.experimental.pallas import tpu_sc as plsc

VOCAB = 1000
EMBED = 128
BATCH = 16384
SEQ = 20

NC = 2
NS = 16
LANES = 16
NW = NC * NS

NWORD = EMBED // 2
TPW = BATCH // NW
CH = 16
GR = CH * SEQ
NCHUNK = TPW // CH


def _fold_body(emb_ref, w_ref, b_ref, out_ref):
    tbl = emb_ref[...]
    rid = lax.broadcasted_iota(jnp.int32, tbl.shape, 0)
    tbl = jnp.where(rid == 0, 0.0, tbl)
    t2 = lax.dot_general(tbl, w_ref[...], (((1,), (1,)), ((), ())),
                         preferred_element_type=jnp.float32)
    t2 = (t2 + b_ref[...]) * (1.0 / SEQ)
    bits = pltpu.bitcast(t2, jnp.int32) + 0x8000
    pair = bits.reshape(VOCAB, NWORD, 2)
    packed = jnp.bitwise_or(
        lax.shift_right_logical(pair[:, :, 0], 16),
        jnp.bitwise_and(pair[:, :, 1], jnp.int32(-65536)))
    out_ref[...] = packed


_fold = pl.pallas_call(
    _fold_body,
    out_shape=jax.ShapeDtypeStruct((VOCAB, NWORD), jnp.int32),
)


def _gather_body(t2_hbm, tok_hbm, out_hbm, tok_v, buf_v, stage_v, sems):
    c = lax.axis_index("c")
    s = lax.axis_index("s")
    w = s * NC + c
    pltpu.sync_copy(tok_hbm.at[pl.ds(w * (TPW * SEQ), TPW * SEQ)], tok_v)

    iota2 = lax.iota(jnp.int32, LANES) * 2

    def _gather_dma(i, buf):
        return pltpu.async_copy(
            t2_hbm.at[tok_v.at[pl.ds(i * GR, GR)]], buf_v.at[buf],
            sems.at[buf])

    def _drain(buf):
        pltpu.make_async_copy(t2_hbm.at[pl.ds(0, GR)], buf_v.at[buf],
                              sems.at[buf]).wait()

    _gather_dma(0, 0)
    _gather_dma(1, 1)

    def th_body(th, carry):
        for ii in range(2):
            i = th * 2 + ii
            _drain(ii)

            @pl.when(th < (NCHUNK // 2) - 1)
            def _next(i=i, ii=ii):
                _gather_dma(i + 2, ii)

            def r_body(r, carry2, ii=ii):
                accs = [plsc.bitcast(
                            buf_v[ii, r * SEQ, pl.ds(k * LANES, LANES)],
                            jnp.bfloat16)
                        for k in range(4)]
                for l in range(1, SEQ):
                    for k in range(4):
                        accs[k] = accs[k] + plsc.bitcast(
                            buf_v[ii, r * SEQ + l, pl.ds(k * LANES, LANES)],
                            jnp.bfloat16)
                out_base = i * (CH * EMBED) + r * EMBED
                for k in range(4):
                    a = plsc.bitcast(accs[k], jnp.int32)
                    lo = plsc.bitcast(lax.shift_left(a, 16), jnp.float32)
                    hi = plsc.bitcast(
                        jnp.bitwise_and(a, jnp.int32(-65536)), jnp.float32)
                    idx = iota2 + (out_base + k * 2 * LANES)
                    plsc.store_scatter(stage_v, [idx], lo)
                    plsc.store_scatter(stage_v, [idx + 1], hi)
                return carry2

            lax.fori_loop(0, CH, r_body, 0)
        return carry

    lax.fori_loop(0, NCHUNK // 2, th_body, 0)
    pltpu.sync_copy(stage_v,
                    out_hbm.at[pl.ds(w * (TPW * EMBED), TPW * EMBED)])


_gather = functools.partial(
    pl.kernel,
    out_type=jax.ShapeDtypeStruct((BATCH * EMBED,), jnp.float32),
    mesh=plsc.VectorSubcoreMesh(core_axis_name="c", subcore_axis_name="s",
                                num_cores=NC, num_subcores=NS),
    scratch_types=[
        pltpu.VMEM((TPW * SEQ,), jnp.int32),
        pltpu.VMEM((2, GR, NWORD), jnp.int32),
        pltpu.VMEM((TPW * EMBED,), jnp.float32),
        pltpu.SemaphoreType.DMA((2,)),
    ],
    compiler_params=pltpu.CompilerParams(needs_layout_passes=False, use_tc_tiling_on_sc=False),
)(_gather_body)


def kernel(tokens, emb_table, W, b):
    packed = _fold(emb_table, W, b.reshape(1, EMBED))
    out = _gather(packed, tokens.astype(jnp.int32).reshape(-1))
    return out.reshape(BATCH, EMBED)


# speedup vs baseline: 12.3897x; 1.2073x over previous
import functools

import jax
import jax.numpy as jnp
from jax import lax
from jax.experimental import pallas as pl
from jax.experimental.pallas import tpu as pltpu
from jax.experimental.pallas import tpu_sc as plsc

VOCAB = 1000
EMBED = 128
BATCH = 16384
SEQ = 20

NC = 2
NS = 16
LANES = 16
NW = NC * NS

NWORD = EMBED // 2
TPW = BATCH // NW
CH = 16
GR = CH * SEQ
NCHUNK = TPW // CH


def _fold_body(emb_ref, w_ref, b_ref, out_ref):
    tbl = emb_ref[...]
    rid = lax.broadcasted_iota(jnp.int32, tbl.shape, 0)
    tbl = jnp.where(rid == 0, 0.0, tbl)
    t2 = lax.dot_general(tbl, w_ref[...], (((1,), (1,)), ((), ())),
                         preferred_element_type=jnp.float32)
    t2 = (t2 + b_ref[...]) * (1.0 / SEQ)
    bits = pltpu.bitcast(t2, jnp.int32) + 0x8000
    packed = jnp.bitwise_or(
        lax.shift_right_logical(bits[:, :NWORD], 16),
        jnp.bitwise_and(bits[:, NWORD:], jnp.int32(-65536)))
    out_ref[...] = packed


_fold = pl.pallas_call(
    _fold_body,
    out_shape=jax.ShapeDtypeStruct((VOCAB, NWORD), jnp.int32),
)


def _gather_body(t2_hbm, tok_hbm, out_hbm, tok_v, buf_v, stage_v, sems):
    c = lax.axis_index("c")
    s = lax.axis_index("s")
    w = s * NC + c
    pltpu.sync_copy(tok_hbm.at[pl.ds(w * (TPW * SEQ), TPW * SEQ)], tok_v)

    def _gather_dma(i, buf):
        return pltpu.async_copy(
            t2_hbm.at[tok_v.at[pl.ds(i * GR, GR)]], buf_v.at[buf],
            sems.at[buf])

    def _drain(buf):
        pltpu.make_async_copy(t2_hbm.at[pl.ds(0, GR)], buf_v.at[buf],
                              sems.at[buf]).wait()

    _gather_dma(0, 0)
    _gather_dma(1, 1)

    def th_body(th, carry):
        for ii in range(2):
            i = th * 2 + ii
            _drain(ii)

            @pl.when(th < (NCHUNK // 2) - 1)
            def _next(i=i, ii=ii):
                _gather_dma(i + 2, ii)

            def r_body(r, carry2, ii=ii):
                accs = [plsc.bitcast(
                            buf_v[ii, r * SEQ, pl.ds(k * LANES, LANES)],
                            jnp.bfloat16)
                        for k in range(4)]
                for l in range(1, SEQ):
                    for k in range(4):
                        accs[k] = accs[k] + plsc.bitcast(
                            buf_v[ii, r * SEQ + l, pl.ds(k * LANES, LANES)],
                            jnp.bfloat16)
                out_base = i * (CH * EMBED) + r * EMBED
                for k in range(4):
                    a = plsc.bitcast(accs[k], jnp.int32)
                    lo = plsc.bitcast(lax.shift_left(a, 16), jnp.float32)
                    hi = plsc.bitcast(
                        jnp.bitwise_and(a, jnp.int32(-65536)), jnp.float32)
                    stage_v[pl.ds(out_base + k * LANES, LANES)] = lo
                    stage_v[pl.ds(out_base + NWORD + k * LANES, LANES)] = hi
                return carry2

            lax.fori_loop(0, CH, r_body, 0)
        return carry

    lax.fori_loop(0, NCHUNK // 2, th_body, 0)
    pltpu.sync_copy(stage_v,
                    out_hbm.at[pl.ds(w * (TPW * EMBED), TPW * EMBED)])


_gather = functools.partial(
    pl.kernel,
    out_type=jax.ShapeDtypeStruct((BATCH * EMBED,), jnp.float32),
    mesh=plsc.VectorSubcoreMesh(core_axis_name="c", subcore_axis_name="s",
                                num_cores=NC, num_subcores=NS),
    scratch_types=[
        pltpu.VMEM((TPW * SEQ,), jnp.int32),
        pltpu.VMEM((2, GR, NWORD), jnp.int32),
        pltpu.VMEM((TPW * EMBED,), jnp.float32),
        pltpu.SemaphoreType.DMA((2,)),
    ],
    compiler_params=pltpu.CompilerParams(needs_layout_passes=False, use_tc_tiling_on_sc=False),
)(_gather_body)


def kernel(tokens, emb_table, W, b):
    packed = _fold(emb_table, W, b.reshape(1, EMBED))
    out = _gather(packed, tokens.astype(jnp.int32).reshape(-1))
    return out.reshape(BATCH, EMBED)
